# baseline (speedup 1.0000x reference)
import jax
import jax.numpy as jnp
from jax.experimental import pallas as pl
from jax.experimental.pallas import tpu as pltpu

B, J = 16, 8
D, E, R = 64, 64, 96
SB = 8


def _body(alpha_ref, maskT_ref, x_ref, o_ref):
    b = pl.program_id(0)
    logits = maskT_ref[:] * alpha_ref[0, 0]
    m = jnp.max(logits, axis=0, keepdims=True)
    e = jnp.exp(logits - m)
    wT = e / jnp.sum(e, axis=0, keepdims=True)
    lane = jax.lax.broadcasted_iota(jnp.int32, (J, B), 1)
    wcol = jnp.sum(jnp.where(lane == b, wT, 0.0), axis=1)
    x = x_ref[0]
    acc = x[0] * wcol[0]
    for j in range(1, J):
        acc += x[j] * wcol[j]
    o_ref[0] = acc


def kernel(inps, mask, alpha):
    x = inps.transpose(0, 1, 3, 4, 2)
    maskT = mask.T
    alpha2 = jnp.reshape(alpha, (1, 1))
    out = pl.pallas_call(
        _body,
        grid=(B, D // SB),
        in_specs=[
            pl.BlockSpec(memory_space=pltpu.SMEM),
            pl.BlockSpec((J, B), lambda b, d: (0, 0)),
            pl.BlockSpec((1, J, SB, E, R), lambda b, d: (b, 0, d, 0, 0)),
        ],
        out_specs=pl.BlockSpec((1, SB, E, R), lambda b, d: (b, d, 0, 0)),
        out_shape=jax.ShapeDtypeStruct((B, D, E, R), jnp.float32),
        compiler_params=pltpu.CompilerParams(
            dimension_semantics=("parallel", "arbitrary"),
        ),
    )(alpha2, maskT, x)
    sampled = out.transpose(0, 3, 1, 2)
    logp = jnp.zeros((B,), jnp.float32)
    return (sampled, logp)


# speedup vs baseline: 1.0882x; 1.0882x over previous
import functools
import jax
import jax.numpy as jnp
from jax import lax
from jax.experimental import pallas as pl
from jax.experimental.pallas import tpu as pltpu
from jax.experimental.pallas import tpu_sc as plsc

B, J = 16, 8
D, E, R = 64, 64, 96
HE = E // 2
NCH = D
NBUF = 3

_mesh = plsc.VectorSubcoreMesh(core_axis_name="c", subcore_axis_name="s")


@functools.partial(
    pl.kernel,
    out_type=jax.ShapeDtypeStruct((B, D, E, R), jnp.float32),
    mesh=_mesh,
    scratch_types=[
        pltpu.VMEM((J, B), jnp.float32),
        pltpu.VMEM((B,), jnp.float32),
        pltpu.VMEM((J * B + B,), jnp.float32),
        pltpu.VMEM((NBUF, J, HE, R), jnp.float32),
        pltpu.VMEM((NBUF, HE, R), jnp.float32),
        pltpu.SemaphoreType.DMA((NBUF,)),
        pltpu.SemaphoreType.DMA((NBUF,)),
    ],
)
def _sc_kernel(x_hbm, maskT_hbm, alpha_hbm, out_hbm,
               mask_v, alpha_v, w_v, in_buf, out_buf, isem, osem):
    wid = lax.axis_index("s") * 2 + lax.axis_index("c")
    b = wid // 2
    h = wid % 2
    e0 = h * HE

    pltpu.sync_copy(maskT_hbm, mask_v)
    pltpu.sync_copy(alpha_hbm, alpha_v)
    av = alpha_v[...]
    logits = [mask_v[j] * av for j in range(J)]
    mx = logits[0]
    for j in range(1, J):
        mx = jnp.maximum(mx, logits[j])
    es = [jnp.exp(l - mx) for l in logits]
    den = es[0]
    for j in range(1, J):
        den = den + es[j]
    for j in range(J):
        w_v[pl.ds(j * B, B)] = es[j] / den
    wv = [jnp.full((B,), w_v[pl.ds(j * B + b, B)][0]) for j in range(J)]

    def issue_in(s, c):
        pltpu.make_async_copy(
            x_hbm.at[b, :, c, pl.ds(e0, HE), :],
            in_buf.at[s], isem.at[s],
        ).start()

    def wait_in(s, c):
        pltpu.make_async_copy(
            x_hbm.at[b, :, c, pl.ds(e0, HE), :],
            in_buf.at[s], isem.at[s],
        ).wait()

    def issue_out(s, c):
        pltpu.make_async_copy(
            out_buf.at[s], out_hbm.at[b, c, pl.ds(e0, HE), :],
            osem.at[s],
        ).start()

    def wait_out(s):
        pltpu.make_async_copy(
            out_buf.at[s], out_hbm.at[b, 0, pl.ds(e0, HE), :],
            osem.at[s],
        ).wait()

    for s in range(NBUF):
        issue_in(s, jnp.int32(s))

    def rounds(it, carry):
        c0 = it * NBUF
        for s in range(NBUF):
            c = c0 + s
            wait_in(s, c)

            @pl.when(c >= NBUF)
            def _():
                wait_out(s)

            def body(i, carry2):
                for q in range(R // 16):
                    d = pl.ds(q * 16, 16)
                    v = wv[0] * in_buf[s, 0, i, d]
                    for j in range(1, J):
                        v = v + wv[j] * in_buf[s, j, i, d]
                    out_buf[s, i, d] = v
                return carry2

            lax.fori_loop(0, HE, body, 0)
            issue_out(s, c)

            @pl.when(c + NBUF < NCH)
            def _():
                issue_in(s, c + NBUF)
        return carry

    lax.fori_loop(0, NCH // NBUF, rounds, 0)

    for c in range((NCH // NBUF) * NBUF, NCH):
        s = c % NBUF
        wait_in(s, jnp.int32(c))
        wait_out(s)

        def tail_body(i, carry2, _s=s):
            for q in range(R // 16):
                d = pl.ds(q * 16, 16)
                v = wv[0] * in_buf[_s, 0, i, d]
                for j in range(1, J):
                    v = v + wv[j] * in_buf[_s, j, i, d]
                out_buf[_s, i, d] = v
            return carry2

        lax.fori_loop(0, HE, tail_body, 0)
        issue_out(s, jnp.int32(c))

    for s in range(NBUF):
        wait_out(s)


def kernel(inps, mask, alpha):
    x = inps.transpose(0, 1, 3, 4, 2)
    maskT = mask.T
    alpha16 = jnp.full((B,), alpha, dtype=jnp.float32)
    out = _sc_kernel(x, maskT, alpha16)
    sampled = out.transpose(0, 3, 1, 2)
    logp = jnp.zeros((B,), jnp.float32)
    return (sampled, logp)
